# Initial kernel scaffold; baseline (speedup 1.0000x reference)
import jax
import jax.numpy as jnp
from jax import lax
from jax.experimental import pallas as pl
from jax.experimental.pallas import tpu as pltpu

_B, _NIN, _NMID, _NOUT, _FI = 2048, 1024, 2048, 1024, 64


def _densify_body(idx_ref, w_ref, s_ref):
    idx = idx_ref[...]
    w = w_ref[...]
    jcol = lax.broadcasted_iota(jnp.int32, (1, _NMID), 1)
    acc = jnp.zeros(s_ref.shape, jnp.float32)
    for k in range(_FI):
        acc += jnp.where(idx[:, k:k + 1] == jcol, w[:, k:k + 1], 0.0)
    s_ref[...] = acc


def _net_body(xt_ref, win_ref, bin_ref, s0_ref, b0_ref, s1_ref, b1_ref,
              wout_ref, bout_ref, out_ref):
    h = jnp.dot(win_ref[...], xt_ref[...], preferred_element_type=jnp.float32)
    h = jnp.maximum(h + bin_ref[...], 0.0)
    h = jnp.dot(s0_ref[...], h, preferred_element_type=jnp.float32)
    h = jnp.maximum(h + b0_ref[...], 0.0)
    h = jnp.dot(s1_ref[...], h, preferred_element_type=jnp.float32)
    h = jnp.maximum(h + b1_ref[...], 0.0)
    h = jnp.dot(wout_ref[...], h, preferred_element_type=jnp.float32)
    out_ref[...] = h + bout_ref[...]


def _densify(idx, w):
    tm = 256
    return pl.pallas_call(
        _densify_body,
        grid=(idx.shape[0] // tm,),
        in_specs=[
            pl.BlockSpec((tm, _FI), lambda i: (i, 0)),
            pl.BlockSpec((tm, _FI), lambda i: (i, 0)),
        ],
        out_specs=pl.BlockSpec((tm, _NMID), lambda i: (i, 0)),
        out_shape=jax.ShapeDtypeStruct((idx.shape[0], _NMID), jnp.float32),
    )(idx, w)


def kernel(x, W_in, b_in, W0, b0, W1, b1, W_out, b_out, indx0, indx1):
    idx = jnp.concatenate([indx0, indx1], axis=0)
    w = jnp.concatenate([W0, W1], axis=0)
    s_cat = _densify(idx, w)

    tb = 256
    full = lambda r, c: pl.BlockSpec((r, c), lambda i: (0, 0))
    out_t = pl.pallas_call(
        _net_body,
        grid=(_B // tb,),
        in_specs=[
            pl.BlockSpec((_NIN, tb), lambda i: (0, i)),
            full(_NMID, _NIN),
            full(_NMID, 1),
            pl.BlockSpec((_NMID, _NMID), lambda i: (0, 0)),
            full(_NMID, 1),
            pl.BlockSpec((_NMID, _NMID), lambda i: (1, 0)),
            full(_NMID, 1),
            full(_NOUT, _NMID),
            full(_NOUT, 1),
        ],
        out_specs=pl.BlockSpec((_NOUT, tb), lambda i: (0, i)),
        out_shape=jax.ShapeDtypeStruct((_NOUT, _B), jnp.float32),
    )(x.T, W_in, b_in.reshape(-1, 1), s_cat, b0.reshape(-1, 1),
      s_cat, b1.reshape(-1, 1), W_out, b_out.reshape(-1, 1))
    return out_t.T


# speedup vs baseline: 7.5497x; 7.5497x over previous
import jax
import jax.numpy as jnp
from jax import lax
from jax.experimental import pallas as pl
from jax.experimental.pallas import tpu as pltpu

_B, _NIN, _NMID, _NOUT, _FI = 2048, 1024, 2048, 1024, 64
_TB = 512


def _densify_body(idx_ref, w_ref, s_ref):
    idx = idx_ref[...]
    w = w_ref[...]
    jcol = lax.broadcasted_iota(jnp.int32, (1, _NMID), 1)
    acc = jnp.zeros(s_ref.shape, jnp.float32)
    for k in range(_FI):
        acc += jnp.where(idx[:, k:k + 1] == jcol, w[:, k:k + 1], 0.0)
    s_ref[...] = acc


def _densify(idx, w):
    tm = 256
    return pl.pallas_call(
        _densify_body,
        grid=(idx.shape[0] // tm,),
        in_specs=[
            pl.BlockSpec((tm, _FI), lambda i: (i, 0)),
            pl.BlockSpec((tm, _FI), lambda i: (i, 0)),
        ],
        out_specs=pl.BlockSpec((tm, _NMID), lambda i: (i, 0)),
        out_shape=jax.ShapeDtypeStruct((idx.shape[0], _NMID), jnp.float32),
    )(idx, w)


def _stage_a_body(xt_ref, win_ref, bin_ref, s0_ref, b0_ref, out_ref):
    h = jnp.dot(win_ref[...], xt_ref[...], preferred_element_type=jnp.float32)
    h = jnp.maximum(h + bin_ref[...], 0.0)
    h = jnp.dot(s0_ref[...], h, preferred_element_type=jnp.float32)
    out_ref[...] = jnp.maximum(h + b0_ref[...], 0.0)


def _stage_b_body(ht_ref, s1_ref, b1_ref, wout_ref, bout_ref, out_ref):
    h = jnp.dot(s1_ref[...], ht_ref[...], preferred_element_type=jnp.float32)
    h = jnp.maximum(h + b1_ref[...], 0.0)
    h = jnp.dot(wout_ref[...], h, preferred_element_type=jnp.float32)
    out_ref[...] = h + bout_ref[...]


def kernel(x, W_in, b_in, W0, b0, W1, b1, W_out, b_out, indx0, indx1):
    idx = jnp.concatenate([indx0, indx1], axis=0)
    w = jnp.concatenate([W0, W1], axis=0)
    s_cat = _densify(idx, w)

    h2t = pl.pallas_call(
        _stage_a_body,
        grid=(_B // _TB,),
        in_specs=[
            pl.BlockSpec((_NIN, _TB), lambda i: (0, i)),
            pl.BlockSpec((_NMID, _NIN), lambda i: (0, 0)),
            pl.BlockSpec((_NMID, 1), lambda i: (0, 0)),
            pl.BlockSpec((_NMID, _NMID), lambda i: (0, 0)),
            pl.BlockSpec((_NMID, 1), lambda i: (0, 0)),
        ],
        out_specs=pl.BlockSpec((_NMID, _TB), lambda i: (0, i)),
        out_shape=jax.ShapeDtypeStruct((_NMID, _B), jnp.float32),
    )(x.T, W_in, b_in.reshape(-1, 1), s_cat, b0.reshape(-1, 1))

    out_t = pl.pallas_call(
        _stage_b_body,
        grid=(_B // _TB,),
        in_specs=[
            pl.BlockSpec((_NMID, _TB), lambda i: (0, i)),
            pl.BlockSpec((_NMID, _NMID), lambda i: (1, 0)),
            pl.BlockSpec((_NMID, 1), lambda i: (0, 0)),
            pl.BlockSpec((_NOUT, _NMID), lambda i: (0, 0)),
            pl.BlockSpec((_NOUT, 1), lambda i: (0, 0)),
        ],
        out_specs=pl.BlockSpec((_NOUT, _TB), lambda i: (0, i)),
        out_shape=jax.ShapeDtypeStruct((_NOUT, _B), jnp.float32),
    )(h2t, s_cat, b1.reshape(-1, 1), W_out, b_out.reshape(-1, 1))
    return out_t.T


# speedup vs baseline: 14.9081x; 1.9747x over previous
import functools

import jax
import jax.numpy as jnp
from jax import lax
from jax.experimental import pallas as pl
from jax.experimental.pallas import tpu as pltpu
from jax.experimental.pallas import tpu_sc as plsc

_B, _NIN, _NMID, _NOUT, _FI = 2048, 1024, 2048, 1024, 64
_TB = 512
_NW = 32
_ROWS = 2 * _NMID
_RPW = _ROWS // _NW
_CH = 16


def _scatter_body(idx_hbm, w_hbm, zeros_hbm, out_hbm, idx_v, w_v, buf):
    wid = lax.axis_index("s") * 2 + lax.axis_index("c")
    base = wid * _RPW
    pltpu.sync_copy(idx_hbm.at[pl.ds(base, _RPW)], idx_v)
    pltpu.sync_copy(w_hbm.at[pl.ds(base, _RPW)], w_v)

    for c in range(_RPW // _CH):
        pltpu.sync_copy(zeros_hbm, buf)
        for r in range(_CH):
            row = c * _CH + r
            rvec = jnp.full((16,), r, jnp.int32)
            for t in range(_FI // 16):
                iv = idx_v[row, pl.ds(t * 16, 16)]
                wv = w_v[row, pl.ds(t * 16, 16)]
                plsc.addupdate_scatter(buf, [rvec, iv], wv)
        pltpu.sync_copy(buf, out_hbm.at[pl.ds(base + c * _CH, _CH)])


def _densify(idx, w):
    zeros = jnp.zeros((_CH, _NMID), jnp.float32)
    mesh = plsc.VectorSubcoreMesh(core_axis_name="c", subcore_axis_name="s")
    return pl.kernel(
        _scatter_body,
        mesh=mesh,
        compiler_params=pltpu.CompilerParams(needs_layout_passes=False),
        out_type=jax.ShapeDtypeStruct((_ROWS, _NMID), jnp.float32),
        scratch_types=[
            pltpu.VMEM((_RPW, _FI), jnp.int32),
            pltpu.VMEM((_RPW, _FI), jnp.float32),
            pltpu.VMEM((_CH, _NMID), jnp.float32),
        ],
    )(idx, w, zeros)


def _stage_a_body(xt_ref, win_ref, bin_ref, s0_ref, b0_ref, out_ref):
    h = jnp.dot(win_ref[...], xt_ref[...], preferred_element_type=jnp.float32)
    h = jnp.maximum(h + bin_ref[...], 0.0)
    h = jnp.dot(s0_ref[...], h, preferred_element_type=jnp.float32)
    out_ref[...] = jnp.maximum(h + b0_ref[...], 0.0)


def _stage_b_body(ht_ref, s1_ref, b1_ref, wout_ref, bout_ref, out_ref):
    h = jnp.dot(s1_ref[...], ht_ref[...], preferred_element_type=jnp.float32)
    h = jnp.maximum(h + b1_ref[...], 0.0)
    h = jnp.dot(wout_ref[...], h, preferred_element_type=jnp.float32)
    out_ref[...] = h + bout_ref[...]


def kernel(x, W_in, b_in, W0, b0, W1, b1, W_out, b_out, indx0, indx1):
    idx = jnp.concatenate([indx0, indx1], axis=0)
    w = jnp.concatenate([W0, W1], axis=0)
    s_cat = _densify(idx, w)

    h2t = pl.pallas_call(
        _stage_a_body,
        grid=(_B // _TB,),
        in_specs=[
            pl.BlockSpec((_NIN, _TB), lambda i: (0, i)),
            pl.BlockSpec((_NMID, _NIN), lambda i: (0, 0)),
            pl.BlockSpec((_NMID, 1), lambda i: (0, 0)),
            pl.BlockSpec((_NMID, _NMID), lambda i: (0, 0)),
            pl.BlockSpec((_NMID, 1), lambda i: (0, 0)),
        ],
        out_specs=pl.BlockSpec((_NMID, _TB), lambda i: (0, i)),
        out_shape=jax.ShapeDtypeStruct((_NMID, _B), jnp.float32),
    )(x.T, W_in, b_in.reshape(-1, 1), s_cat, b0.reshape(-1, 1))

    out_t = pl.pallas_call(
        _stage_b_body,
        grid=(_B // _TB,),
        in_specs=[
            pl.BlockSpec((_NMID, _TB), lambda i: (0, i)),
            pl.BlockSpec((_NMID, _NMID), lambda i: (1, 0)),
            pl.BlockSpec((_NMID, 1), lambda i: (0, 0)),
            pl.BlockSpec((_NOUT, _NMID), lambda i: (0, 0)),
            pl.BlockSpec((_NOUT, 1), lambda i: (0, 0)),
        ],
        out_specs=pl.BlockSpec((_NOUT, _TB), lambda i: (0, i)),
        out_shape=jax.ShapeDtypeStruct((_NOUT, _B), jnp.float32),
    )(h2t, s_cat, b1.reshape(-1, 1), W_out, b_out.reshape(-1, 1))
    return out_t.T


# speedup vs baseline: 20.2907x; 1.3610x over previous
import functools

import jax
import jax.numpy as jnp
from jax import lax
from jax.experimental import pallas as pl
from jax.experimental.pallas import tpu as pltpu
from jax.experimental.pallas import tpu_sc as plsc

_B, _NIN, _NMID, _NOUT, _FI = 2048, 1024, 2048, 1024, 64
_TB = 512
_NW = 32
_ROWS = 2 * _NMID
_RPW = _ROWS // _NW
_CH = 16


def _scatter_body(idx_hbm, w_hbm, zeros_hbm, out_hbm, idx_v, w_v,
                  buf_a, buf_b, sem_a, sem_b):
    wid = lax.axis_index("s") * 2 + lax.axis_index("c")
    base = wid * _RPW
    pltpu.sync_copy(idx_hbm.at[pl.ds(base, _RPW)], idx_v)
    pltpu.sync_copy(w_hbm.at[pl.ds(base, _RPW)], w_v)
    pltpu.sync_copy(zeros_hbm, buf_a)
    pltpu.sync_copy(zeros_hbm, buf_b)

    bufs = (buf_a, buf_b)
    sems = (sem_a, sem_b)
    z16 = jnp.zeros((16,), jnp.float32)
    dmas = [None, None]
    nchunks = _RPW // _CH
    for c in range(nchunks):
        b = c % 2
        buf = bufs[b]
        if dmas[b] is not None:
            dmas[b].wait()
            for r in range(_CH):
                row = (c - 2) * _CH + r
                rvec = jnp.full((16,), r, jnp.int32)
                for t in range(_FI // 16):
                    iv = idx_v[row, pl.ds(t * 16, 16)]
                    plsc.store_scatter(buf, [rvec, iv], z16)
        for r in range(_CH):
            row = c * _CH + r
            rvec = jnp.full((16,), r, jnp.int32)
            for t in range(_FI // 16):
                iv = idx_v[row, pl.ds(t * 16, 16)]
                wv = w_v[row, pl.ds(t * 16, 16)]
                plsc.addupdate_scatter(buf, [rvec, iv], wv)
        dmas[b] = pltpu.async_copy(
            buf, out_hbm.at[pl.ds(base + c * _CH, _CH)], sems[b])
    dmas[(nchunks - 2) % 2].wait()
    dmas[(nchunks - 1) % 2].wait()


def _densify(idx, w):
    zeros = jnp.zeros((_CH, _NMID), jnp.float32)
    mesh = plsc.VectorSubcoreMesh(core_axis_name="c", subcore_axis_name="s")
    return pl.kernel(
        _scatter_body,
        mesh=mesh,
        compiler_params=pltpu.CompilerParams(needs_layout_passes=False),
        out_type=jax.ShapeDtypeStruct((_ROWS, _NMID), jnp.float32),
        scratch_types=[
            pltpu.VMEM((_RPW, _FI), jnp.int32),
            pltpu.VMEM((_RPW, _FI), jnp.float32),
            pltpu.VMEM((_CH, _NMID), jnp.float32),
            pltpu.VMEM((_CH, _NMID), jnp.float32),
            pltpu.SemaphoreType.DMA,
            pltpu.SemaphoreType.DMA,
        ],
    )(idx, w, zeros)


def _stage_a_body(x_ref, win_ref, bin_ref, s0_ref, b0_ref, out_ref):
    h = lax.dot_general(win_ref[...], x_ref[...],
                        (((1,), (1,)), ((), ())),
                        preferred_element_type=jnp.float32)
    h = jnp.maximum(h + bin_ref[...], 0.0)
    h = jnp.dot(s0_ref[...], h, preferred_element_type=jnp.float32)
    out_ref[...] = jnp.maximum(h + b0_ref[...], 0.0)


def _stage_b_body(ht_ref, s1_ref, b1_ref, wout_ref, bout_ref, out_ref):
    h = jnp.dot(s1_ref[...], ht_ref[...], preferred_element_type=jnp.float32)
    h = jnp.maximum(h + b1_ref[...], 0.0)
    o = lax.dot_general(h, wout_ref[...],
                        (((0,), (1,)), ((), ())),
                        preferred_element_type=jnp.float32)
    out_ref[...] = o + bout_ref[...]


def kernel(x, W_in, b_in, W0, b0, W1, b1, W_out, b_out, indx0, indx1):
    idx = jnp.concatenate([indx0, indx1], axis=0)
    w = jnp.concatenate([W0, W1], axis=0)
    s_cat = _densify(idx, w)

    h2t = pl.pallas_call(
        _stage_a_body,
        grid=(_B // _TB,),
        in_specs=[
            pl.BlockSpec((_TB, _NIN), lambda i: (i, 0)),
            pl.BlockSpec((_NMID, _NIN), lambda i: (0, 0)),
            pl.BlockSpec((_NMID, 1), lambda i: (0, 0)),
            pl.BlockSpec((_NMID, _NMID), lambda i: (0, 0)),
            pl.BlockSpec((_NMID, 1), lambda i: (0, 0)),
        ],
        out_specs=pl.BlockSpec((_NMID, _TB), lambda i: (0, i)),
        out_shape=jax.ShapeDtypeStruct((_NMID, _B), jnp.float32),
    )(x, W_in, b_in.reshape(-1, 1), s_cat, b0.reshape(-1, 1))

    out = pl.pallas_call(
        _stage_b_body,
        grid=(_B // _TB,),
        in_specs=[
            pl.BlockSpec((_NMID, _TB), lambda i: (0, i)),
            pl.BlockSpec((_NMID, _NMID), lambda i: (1, 0)),
            pl.BlockSpec((_NMID, 1), lambda i: (0, 0)),
            pl.BlockSpec((_NOUT, _NMID), lambda i: (0, 0)),
            pl.BlockSpec((1, _NOUT), lambda i: (0, 0)),
        ],
        out_specs=pl.BlockSpec((_TB, _NOUT), lambda i: (i, 0)),
        out_shape=jax.ShapeDtypeStruct((_B, _NOUT), jnp.float32),
    )(h2t, s_cat, b1.reshape(-1, 1), W_out, b_out.reshape(1, -1))
    return out
